# baseline (speedup 1.0000x reference)
import functools

import jax
import jax.numpy as jnp
from jax import lax
from jax.experimental import pallas as pl
from jax.experimental.pallas import tpu as pltpu
from jax.experimental.pallas import tpu_sc as plsc

_WORKER_NUM = 1000
_TASK_NUM = 20000
_ABILITY_NUM = 128
_EDGE_TYPE = 2
_K = _TASK_NUM * _EDGE_TYPE

_NEG_BIG = -1e38

_NW = 32
_RPW = 32
_PAD_W = _NW * _RPW
_HALF = _K // 2
_LANES = 16


def _coef_body(wf_ref, w_ref, b_ref, bout_ref, zout_ref):
    z = jnp.dot(wf_ref[...], w_ref[...], preferred_element_type=jnp.float32)
    z = z + b_ref[0, 0]
    p1 = jax.nn.sigmoid(z)
    a = jnp.maximum(jnp.log(p1), _NEG_BIG)
    bl = jnp.maximum(jnp.log(1.0 - p1), _NEG_BIG)
    bout_ref[...] = jnp.broadcast_to(bl, (_PAD_W, _LANES))
    zout_ref[...] = jnp.broadcast_to(a - bl, (_PAD_W, _LANES))


def _coefs(wf_pad, W, b2):
    return pl.pallas_call(
        _coef_body,
        in_specs=[
            pl.BlockSpec((_PAD_W, _ABILITY_NUM), lambda: (0, 0)),
            pl.BlockSpec((_ABILITY_NUM, 1), lambda: (0, 0)),
            pl.BlockSpec((1, 1), lambda: (0, 0)),
        ],
        out_specs=[
            pl.BlockSpec((_PAD_W, _LANES), lambda: (0, 0)),
            pl.BlockSpec((_PAD_W, _LANES), lambda: (0, 0)),
        ],
        out_shape=[
            jax.ShapeDtypeStruct((_PAD_W, _LANES), jnp.float32),
            jax.ShapeDtypeStruct((_PAD_W, _LANES), jnp.float32),
        ],
    )(wf_pad, W, b2)


_SC_MESH = plsc.VectorSubcoreMesh(core_axis_name="c", subcore_axis_name="s")


@functools.partial(
    pl.kernel,
    out_type=jax.ShapeDtypeStruct((_WORKER_NUM, _EDGE_TYPE, _TASK_NUM),
                                  jnp.float32),
    mesh=_SC_MESH,
    scratch_types=[
        pltpu.VMEM((_TASK_NUM,), jnp.float32),
        pltpu.VMEM((_TASK_NUM,), jnp.float32),
        pltpu.VMEM((_RPW * _LANES,), jnp.float32),
        pltpu.VMEM((_RPW * _LANES,), jnp.float32),
        pltpu.VMEM((_EDGE_TYPE, _TASK_NUM), jnp.float32),
        pltpu.VMEM((_EDGE_TYPE, _TASK_NUM), jnp.float32),
        pltpu.SemaphoreType.DMA,
        pltpu.SemaphoreType.DMA,
    ],
)
def _sc_broadcast(tau0_hbm, tau1_hbm, coefb_hbm, coefz_hbm, out_hbm,
                  tau0_v, tau1_v, bv, zv, obuf0, obuf1, sem0, sem1):
    wid = lax.axis_index("s") * 2 + lax.axis_index("c")
    base_row = wid * _RPW
    pltpu.sync_copy(tau0_hbm, tau0_v)
    pltpu.sync_copy(tau1_hbm, tau1_v)
    pltpu.sync_copy(coefb_hbm.at[pl.ds(base_row * _LANES, _RPW * _LANES)], bv)
    pltpu.sync_copy(coefz_hbm.at[pl.ds(base_row * _LANES, _RPW * _LANES)], zv)

    def row_body(r, _):
        row = base_row + r

        @pl.when(row < _WORKER_NUM)
        def _():
            bs = bv[pl.ds(r * _LANES, _LANES)]
            zs = zv[pl.ds(r * _LANES, _LANES)]
            for par, obuf, sem in ((0, obuf0, sem0), (1, obuf1, sem1)):

                @pl.when((r & 1) == par)
                def _():
                    @pl.when(r > 1)
                    def _():
                        pltpu.make_async_copy(obuf, out_hbm.at[0], sem).wait()

                    @plsc.parallel_loop(0, _TASK_NUM // _LANES, unroll=20)
                    def _(i):
                        t0 = tau0_v[pl.ds(i * _LANES, _LANES)]
                        obuf[0, pl.ds(i * _LANES, _LANES)] = jnp.exp(bs + zs * t0)

                    @plsc.parallel_loop(0, _TASK_NUM // _LANES, unroll=20)
                    def _(i):
                        t1 = tau1_v[pl.ds(i * _LANES, _LANES)]
                        obuf[1, pl.ds(i * _LANES, _LANES)] = jnp.exp(bs + zs * t1)

                    pltpu.async_copy(obuf, out_hbm.at[row], sem)
        return 0

    lax.fori_loop(0, _RPW, row_body, 0)
    pltpu.make_async_copy(obuf0, out_hbm.at[0], sem0).wait()
    pltpu.make_async_copy(obuf1, out_hbm.at[0], sem1).wait()


@jax.jit
def kernel(inputs, W, b, P0):
    wf = inputs[:_WORKER_NUM, :_ABILITY_NUM]
    task_feature = inputs[_WORKER_NUM:_WORKER_NUM + _TASK_NUM, :_EDGE_TYPE]
    tau0 = task_feature[:, 0]
    tau1 = task_feature[:, 1]
    wf_pad = jnp.pad(wf, ((0, _PAD_W - _WORKER_NUM), (0, 0)))
    coef_b, coef_z = _coefs(wf_pad, W, b.reshape(1, 1))
    out = _sc_broadcast(tau0, tau1, coef_b.reshape(_PAD_W * _LANES),
                        coef_z.reshape(_PAD_W * _LANES))
    return jnp.transpose(out, (0, 2, 1))


# speedup vs baseline: 1.1294x; 1.1294x over previous
import functools

import jax
import jax.numpy as jnp
from jax import lax
from jax.experimental import pallas as pl
from jax.experimental.pallas import tpu as pltpu
from jax.experimental.pallas import tpu_sc as plsc

_WORKER_NUM = 1000
_TASK_NUM = 20000
_ABILITY_NUM = 128
_EDGE_TYPE = 2
_K = _TASK_NUM * _EDGE_TYPE

_NEG_BIG = -1e38

_NW = 32
_RPW = 32
_PAD_W = _NW * _RPW
_HALF = _K // 2
_LANES = 16


def _coef_body(wf_ref, w_ref, b_ref, bout_ref, zout_ref):
    z = jnp.dot(wf_ref[...], w_ref[...], preferred_element_type=jnp.float32)
    z = z + b_ref[0, 0]
    p1 = jax.nn.sigmoid(z)
    a = jnp.maximum(jnp.log(p1), _NEG_BIG)
    bl = jnp.maximum(jnp.log(1.0 - p1), _NEG_BIG)
    bout_ref[...] = jnp.broadcast_to(bl, (_PAD_W, _LANES))
    zout_ref[...] = jnp.broadcast_to(a - bl, (_PAD_W, _LANES))


def _coefs(wf_pad, W, b2):
    return pl.pallas_call(
        _coef_body,
        in_specs=[
            pl.BlockSpec((_PAD_W, _ABILITY_NUM), lambda: (0, 0)),
            pl.BlockSpec((_ABILITY_NUM, 1), lambda: (0, 0)),
            pl.BlockSpec((1, 1), lambda: (0, 0)),
        ],
        out_specs=[
            pl.BlockSpec((_PAD_W, _LANES), lambda: (0, 0)),
            pl.BlockSpec((_PAD_W, _LANES), lambda: (0, 0)),
        ],
        out_shape=[
            jax.ShapeDtypeStruct((_PAD_W, _LANES), jnp.float32),
            jax.ShapeDtypeStruct((_PAD_W, _LANES), jnp.float32),
        ],
    )(wf_pad, W, b2)


_SC_MESH = plsc.VectorSubcoreMesh(core_axis_name="c", subcore_axis_name="s")


@functools.partial(
    pl.kernel,
    out_type=jax.ShapeDtypeStruct((_WORKER_NUM, _EDGE_TYPE, _TASK_NUM),
                                  jnp.float32),
    mesh=_SC_MESH,
    scratch_types=[
        pltpu.VMEM((_TASK_NUM,), jnp.float32),
        pltpu.VMEM((_TASK_NUM,), jnp.float32),
        pltpu.VMEM((_RPW * _LANES,), jnp.float32),
        pltpu.VMEM((_RPW * _LANES,), jnp.float32),
        pltpu.VMEM((_EDGE_TYPE, _TASK_NUM), jnp.float32),
        pltpu.VMEM((_EDGE_TYPE, _TASK_NUM), jnp.float32),
        pltpu.SemaphoreType.DMA,
        pltpu.SemaphoreType.DMA,
    ],
)
def _sc_broadcast(tau0_hbm, tau1_hbm, coefb_hbm, coefz_hbm, out_hbm,
                  tau0_v, tau1_v, bv, zv, obuf0, obuf1, sem0, sem1):
    wid = lax.axis_index("s") * 2 + lax.axis_index("c")
    base_row = wid * _RPW
    pltpu.sync_copy(tau0_hbm, tau0_v)
    pltpu.sync_copy(tau1_hbm, tau1_v)
    pltpu.sync_copy(coefb_hbm.at[pl.ds(base_row * _LANES, _RPW * _LANES)], bv)
    pltpu.sync_copy(coefz_hbm.at[pl.ds(base_row * _LANES, _RPW * _LANES)], zv)

    def row_body(r, _):
        row = base_row + r

        @pl.when(row < _WORKER_NUM)
        def _():
            bs = bv[pl.ds(r * _LANES, _LANES)]
            zs = zv[pl.ds(r * _LANES, _LANES)]
            for par, obuf, sem in ((0, obuf0, sem0), (1, obuf1, sem1)):

                @pl.when((r & 1) == par)
                def _():
                    @pl.when(r > 1)
                    def _():
                        pltpu.make_async_copy(obuf, out_hbm.at[0], sem).wait()

                    @plsc.parallel_loop(0, _TASK_NUM // _LANES, unroll=16)
                    def _(i):
                        t0 = tau0_v[pl.ds(i * _LANES, _LANES)]
                        obuf[0, pl.ds(i * _LANES, _LANES)] = jnp.exp(bs + zs * t0)

                    @plsc.parallel_loop(0, _TASK_NUM // _LANES, unroll=16)
                    def _(i):
                        t1 = tau1_v[pl.ds(i * _LANES, _LANES)]
                        obuf[1, pl.ds(i * _LANES, _LANES)] = jnp.exp(bs + zs * t1)

                    pltpu.async_copy(obuf, out_hbm.at[row], sem)
        return 0

    lax.fori_loop(0, _RPW, row_body, 0)
    pltpu.make_async_copy(obuf0, out_hbm.at[0], sem0).wait()
    pltpu.make_async_copy(obuf1, out_hbm.at[0], sem1).wait()


@jax.jit
def kernel(inputs, W, b, P0):
    wf = inputs[:_WORKER_NUM, :_ABILITY_NUM]
    task_feature = inputs[_WORKER_NUM:_WORKER_NUM + _TASK_NUM, :_EDGE_TYPE]
    tau0 = task_feature[:, 0]
    tau1 = task_feature[:, 1]
    wf_pad = jnp.pad(wf, ((0, _PAD_W - _WORKER_NUM), (0, 0)))
    coef_b, coef_z = _coefs(wf_pad, W, b.reshape(1, 1))
    out = _sc_broadcast(tau0, tau1, coef_b.reshape(_PAD_W * _LANES),
                        coef_z.reshape(_PAD_W * _LANES))
    return jnp.transpose(out, (0, 2, 1))
